# baseline (speedup 1.0000x reference)
import functools

import jax
import jax.numpy as jnp
from jax import lax
from jax.experimental import pallas as pl
from jax.experimental.pallas import tpu as pltpu
from jax.experimental.pallas import tpu_sc as plsc

IN_DIM = 3
EMB = 32
STEPS_PER_DAY = 288
NC = 2
NS = 16
NW = NC * NS
L = 16
T = 128


def _tec_body(x_hbm, wt_hbm, b_hbm, tod_hbm, dow_hbm, out_hbm,
              xv, tidx, didx, feat, todv, dowv, wv, bv, sem_t, sem_d):
    n_tokens = x_hbm.shape[0] // IN_DIM
    per_worker = n_tokens // NW
    n_chunks = per_worker // T
    wid = lax.axis_index("s") * NC + lax.axis_index("c")
    base0 = wid * per_worker

    pltpu.sync_copy(wt_hbm, wv)
    pltpu.sync_copy(b_hbm, bv)
    w00 = wv[0]
    w01 = wv[1]
    w10 = wv[2]
    w11 = wv[3]
    w20 = wv[4]
    w21 = wv[5]
    b0 = bv[0]
    b1 = bv[1]
    iota = lax.iota(jnp.int32, L)

    def chunk(i, carry):
        base = base0 + i * T
        pltpu.sync_copy(x_hbm.at[pl.ds(base * IN_DIM, T * IN_DIM)],
                        xv.at[pl.ds(0, T * IN_DIM)])

        def idx_group(g, carry):
            rows = g * (L * IN_DIM) + iota * IN_DIM
            x1 = plsc.load_gather(xv, [rows + 1])
            x2 = plsc.load_gather(xv, [rows + 2])
            tidx[pl.ds(g * L, L)] = (x1 * float(STEPS_PER_DAY)).astype(jnp.int32)
            didx[pl.ds(g * L, L)] = x2.astype(jnp.int32)
            return carry
        lax.fori_loop(0, T // L, idx_group, 0, unroll=True)

        cp_t = pltpu.async_copy(tod_hbm.at[tidx], todv, sem_t)
        cp_d = pltpu.async_copy(dow_hbm.at[didx], dowv, sem_d)

        def tok(t, carry):
            v = xv[pl.ds(t * IN_DIM, L)]
            x0 = v[0]
            x1 = v[1]
            x2 = v[2]
            feat[t, pl.ds(0, L)] = w00 * x0 + w10 * x1 + w20 * x2 + b0
            feat[t, pl.ds(L, L)] = w01 * x0 + w11 * x1 + w21 * x2 + b1
            return carry
        lax.fori_loop(0, T, tok, 0)

        cp_t.wait()
        cp_d.wait()
        pltpu.sync_copy(feat, out_hbm.at[pl.ds(base, T), pl.ds(0, EMB)])
        pltpu.sync_copy(todv, out_hbm.at[pl.ds(base, T), pl.ds(EMB, EMB)])
        pltpu.sync_copy(dowv, out_hbm.at[pl.ds(base, T), pl.ds(2 * EMB, EMB)])
        return carry

    lax.fori_loop(0, n_chunks, chunk, 0)


def kernel(x, W, b, tod_table, dow_table):
    batch, steps, nodes, nfeat = x.shape
    n = batch * steps * nodes
    assert n % (NW * T) == 0
    x2 = x.reshape(n * nfeat)
    wt = jnp.transpose(W).reshape(2 * IN_DIM, L)
    b2 = b.reshape(2, L)

    mesh = plsc.VectorSubcoreMesh(core_axis_name="c", subcore_axis_name="s")
    run = pl.kernel(
        _tec_body,
        out_type=jax.ShapeDtypeStruct((n, 3 * EMB), jnp.float32),
        mesh=mesh,
        compiler_params=pltpu.CompilerParams(needs_layout_passes=False,
                                             use_tc_tiling_on_sc=False),
        scratch_types=[
            pltpu.VMEM((T * IN_DIM + 128,), jnp.float32),
            pltpu.VMEM((T,), jnp.int32),
            pltpu.VMEM((T,), jnp.int32),
            pltpu.VMEM((T, EMB), jnp.float32),
            pltpu.VMEM((T, EMB), jnp.float32),
            pltpu.VMEM((T, EMB), jnp.float32),
            pltpu.VMEM((2 * IN_DIM, L), jnp.float32),
            pltpu.VMEM((2, L), jnp.float32),
            pltpu.SemaphoreType.DMA,
            pltpu.SemaphoreType.DMA,
        ],
    )
    out = run(x2, wt, b2, tod_table, dow_table)
    return out.reshape(batch, steps, nodes, 3 * EMB)


# speedup vs baseline: 11.3528x; 11.3528x over previous
import jax
import jax.numpy as jnp
from jax import lax
from jax.experimental import pallas as pl
from jax.experimental.pallas import tpu as pltpu
from jax.experimental.pallas import tpu_sc as plsc

IN_DIM = 3
EMB = 32
STEPS_PER_DAY = 288
DOW = 7
NC = 2
NS = 16
NW = NC * NS
L = 16

B, S, V = 64, 12, 1024
BT, BR = 8, 8
VT, VC = 8, 128
DT, DR = 12, 8
SLABS = S * BT
SLABS_PER_W = SLABS // NW


def _tec_body(x6, wb_hbm, bb_hbm, todf_hbm, dowf_hbm, out6,
              xb0, xb1, xb2, tmap, dmap, blk0, blk1,
              wv, bv, todv, dowv, semx, semt, sem0, sem1):
    wid = lax.axis_index("s") * NC + lax.axis_index("c")

    cp_tab = [
        pltpu.async_copy(wb_hbm, wv, semt),
        pltpu.async_copy(bb_hbm, bv, semt),
        pltpu.async_copy(todf_hbm, todv, semt),
        pltpu.async_copy(dowf_hbm, dowv, semt),
    ]
    for cp in cp_tab:
        cp.wait()

    iota = lax.iota(jnp.int32, L)
    blks = [blk0, blk1]
    sems = [sem0, sem1]

    def slab_body(k, carry):
        slab = wid * SLABS_PER_W + k
        s = slab // BT
        bt = slab % BT

        cpx = [
            pltpu.async_copy(x6.at[s, 0, bt], xb0, semx),
            pltpu.async_copy(x6.at[s, 1, bt], xb1, semx),
            pltpu.async_copy(x6.at[s, 2, bt], xb2, semx),
        ]
        for cp in cpx:
            cp.wait()

        def br_body(br, carry):
            b = bt * BR + br

            def idx_body(g2, carry):
                vt = g2 // 8
                g = g2 % 8
                x1v = xb1[vt, br, pl.ds(g * L, L)]
                x2v = xb2[vt, br, pl.ds(g * L, L)]
                ti = (x1v * float(STEPS_PER_DAY)).astype(jnp.int32)
                di = x2v.astype(jnp.int32)
                tmap[pl.ds(g2 * L, L)] = ti * EMB
                dmap[pl.ds(g2 * L, L)] = di * EMB
                return carry
            lax.fori_loop(0, VT * 8, idx_body, 0)

            for dt in range(DT):
                blk = blks[dt % 2]
                if dt >= 2:
                    pltpu.make_async_copy(blk, out6.at[b, s, dt - 2],
                                          sems[dt % 2]).wait()
                if dt < 4:
                    wrows = [[wv[dt * DR + dr, j] for j in range(IN_DIM)]
                             for dr in range(DR)]
                    brows = [bv[dt * DR + dr] for dr in range(DR)]

                    def feat_body(g2, carry):
                        vt = g2 // 8
                        g = g2 % 8
                        x0v = xb0[vt, br, pl.ds(g * L, L)]
                        x1v = xb1[vt, br, pl.ds(g * L, L)]
                        x2v = xb2[vt, br, pl.ds(g * L, L)]
                        for dr in range(DR):
                            w = wrows[dr]
                            f = w[0] * x0v + w[1] * x1v + w[2] * x2v + brows[dr]
                            blk[vt, dr, pl.ds(g * L, L)] = f
                        return carry
                    lax.fori_loop(0, VT * 8, feat_body, 0)
                else:
                    tab = todv if dt < 8 else dowv
                    imap = tmap if dt < 8 else dmap
                    dbase = (dt - 4) * DR if dt < 8 else (dt - 8) * DR

                    def gat_body(g2, carry):
                        vt = g2 // 8
                        g = g2 % 8
                        iv = imap[pl.ds(g2 * L, L)]
                        for dr in range(DR):
                            r = plsc.load_gather(tab, [iv + (dbase + dr)])
                            blk[vt, dr, pl.ds(g * L, L)] = r
                        return carry
                    lax.fori_loop(0, VT * 8, gat_body, 0)

                pltpu.async_copy(blk, out6.at[b, s, dt], sems[dt % 2])

            pltpu.make_async_copy(blks[0], out6.at[b, s, DT - 2], sems[0]).wait()
            pltpu.make_async_copy(blks[1], out6.at[b, s, DT - 1], sems[1]).wait()
            return carry
        lax.fori_loop(0, BR, br_body, 0)
        return carry

    lax.fori_loop(0, SLABS_PER_W, slab_body, 0)


def kernel(x, W, b, tod_table, dow_table):
    x6 = (x.transpose(1, 3, 0, 2)
           .reshape(S, IN_DIM, BT, BR, VT, VC)
           .transpose(0, 1, 2, 4, 3, 5))
    wb = jnp.broadcast_to(W[:, :, None], (EMB, IN_DIM, L))
    bb = jnp.broadcast_to(b[:, None], (EMB, L))
    todf = tod_table.reshape(STEPS_PER_DAY * EMB)
    dowf = dow_table.reshape(DOW * EMB)

    mesh = plsc.VectorSubcoreMesh(core_axis_name="c", subcore_axis_name="s")
    run = pl.kernel(
        _tec_body,
        out_type=jax.ShapeDtypeStruct((B, S, DT, VT, DR, VC), jnp.float32),
        mesh=mesh,
        compiler_params=pltpu.CompilerParams(needs_layout_passes=False,
                                             use_tc_tiling_on_sc=False),
        scratch_types=[
            pltpu.VMEM((VT, BR, VC), jnp.float32),
            pltpu.VMEM((VT, BR, VC), jnp.float32),
            pltpu.VMEM((VT, BR, VC), jnp.float32),
            pltpu.VMEM((V,), jnp.int32),
            pltpu.VMEM((V,), jnp.int32),
            pltpu.VMEM((VT, DR, VC), jnp.float32),
            pltpu.VMEM((VT, DR, VC), jnp.float32),
            pltpu.VMEM((EMB, IN_DIM, L), jnp.float32),
            pltpu.VMEM((EMB, L), jnp.float32),
            pltpu.VMEM((STEPS_PER_DAY * EMB,), jnp.float32),
            pltpu.VMEM((DOW * EMB,), jnp.float32),
            pltpu.SemaphoreType.DMA,
            pltpu.SemaphoreType.DMA,
            pltpu.SemaphoreType.DMA,
            pltpu.SemaphoreType.DMA,
        ],
    )
    out6 = run(x6, wb, bb, todf, dowf)
    return (out6.transpose(0, 1, 3, 5, 2, 4)
                .reshape(B, S, V, IN_DIM * EMB))


# speedup vs baseline: 25.1890x; 2.2187x over previous
import jax
import jax.numpy as jnp
from jax import lax
from jax.experimental import pallas as pl
from jax.experimental.pallas import tpu as pltpu
from jax.experimental.pallas import tpu_sc as plsc

IN_DIM = 3
EMB = 32
STEPS_PER_DAY = 288
DOW = 7
NC = 2
NS = 16
NW = NC * NS
L = 16

B, S, V = 64, 12, 1024
BT, BR = 8, 8
VT, VC = 8, 128
DT, DR = 12, 8
SLABS = S * BT
SLABS_PER_W = SLABS // NW


def _tec_body(x6, wb_hbm, bb_hbm, todf_hbm, dowf_hbm, out6,
              xb0, xb1, xb2, tmap, dmap, blk0, blk1,
              wv, bv, todv, dowv, semx, semt, sem0, sem1):
    wid = lax.axis_index("s") * NC + lax.axis_index("c")

    cp_tab = [
        pltpu.async_copy(wb_hbm, wv, semt),
        pltpu.async_copy(bb_hbm, bv, semt),
        pltpu.async_copy(todf_hbm, todv, semt),
        pltpu.async_copy(dowf_hbm, dowv, semt),
    ]
    for cp in cp_tab:
        cp.wait()

    iota = lax.iota(jnp.int32, L)
    blks = [blk0, blk1]
    sems = [sem0, sem1]

    def slab_body(k, carry):
        slab = wid * SLABS_PER_W + k
        s = slab // BT
        bt = slab % BT

        cpx = [
            pltpu.async_copy(x6.at[s, 0, bt], xb0, semx),
            pltpu.async_copy(x6.at[s, 1, bt], xb1, semx),
            pltpu.async_copy(x6.at[s, 2, bt], xb2, semx),
        ]
        for cp in cpx:
            cp.wait()

        def br_body(br, carry):
            b = bt * BR + br

            @plsc.parallel_loop(0, VT * 8, unroll=2)
            def idx_body(g2):
                vt = g2 // 8
                g = g2 % 8
                x1v = xb1[vt, br, pl.ds(g * L, L)]
                x2v = xb2[vt, br, pl.ds(g * L, L)]
                ti = (x1v * float(STEPS_PER_DAY)).astype(jnp.int32)
                di = x2v.astype(jnp.int32)
                tmap[pl.ds(g2 * L, L)] = ti * EMB
                dmap[pl.ds(g2 * L, L)] = di * EMB

            for dt in range(DT):
                blk = blks[dt % 2]
                if dt >= 2:
                    pltpu.make_async_copy(blk, out6.at[b, s, dt - 2],
                                          sems[dt % 2]).wait()
                if dt < 4:
                    wrows = [[wv[dt * DR + dr, j] for j in range(IN_DIM)]
                             for dr in range(DR)]
                    brows = [bv[dt * DR + dr] for dr in range(DR)]

                    @plsc.parallel_loop(0, VT * 8, unroll=2)
                    def feat_body(g2):
                        vt = g2 // 8
                        g = g2 % 8
                        x0v = xb0[vt, br, pl.ds(g * L, L)]
                        x1v = xb1[vt, br, pl.ds(g * L, L)]
                        x2v = xb2[vt, br, pl.ds(g * L, L)]
                        fs = [w[0] * x0v + w[1] * x1v + w[2] * x2v + bd
                              for w, bd in zip(wrows, brows)]
                        for dr in range(DR):
                            blk[vt, dr, pl.ds(g * L, L)] = fs[dr]
                else:
                    tab = todv if dt < 8 else dowv
                    imap = tmap if dt < 8 else dmap
                    dbase = (dt - 4) * DR if dt < 8 else (dt - 8) * DR

                    @plsc.parallel_loop(0, VT * 8, unroll=2)
                    def gat_body(g2):
                        vt = g2 // 8
                        g = g2 % 8
                        iv = imap[pl.ds(g2 * L, L)]
                        rs = [plsc.load_gather(tab, [iv + (dbase + dr)])
                              for dr in range(DR)]
                        for dr in range(DR):
                            blk[vt, dr, pl.ds(g * L, L)] = rs[dr]

                pltpu.async_copy(blk, out6.at[b, s, dt], sems[dt % 2])

            pltpu.make_async_copy(blks[0], out6.at[b, s, DT - 2], sems[0]).wait()
            pltpu.make_async_copy(blks[1], out6.at[b, s, DT - 1], sems[1]).wait()
            return carry
        lax.fori_loop(0, BR, br_body, 0)
        return carry

    lax.fori_loop(0, SLABS_PER_W, slab_body, 0)


def kernel(x, W, b, tod_table, dow_table):
    x6 = (x.transpose(1, 3, 0, 2)
           .reshape(S, IN_DIM, BT, BR, VT, VC)
           .transpose(0, 1, 2, 4, 3, 5))
    wb = jnp.broadcast_to(W[:, :, None], (EMB, IN_DIM, L))
    bb = jnp.broadcast_to(b[:, None], (EMB, L))
    todf = tod_table.reshape(STEPS_PER_DAY * EMB)
    dowf = dow_table.reshape(DOW * EMB)

    mesh = plsc.VectorSubcoreMesh(core_axis_name="c", subcore_axis_name="s")
    run = pl.kernel(
        _tec_body,
        out_type=jax.ShapeDtypeStruct((B, S, DT, VT, DR, VC), jnp.float32),
        mesh=mesh,
        compiler_params=pltpu.CompilerParams(needs_layout_passes=False,
                                             use_tc_tiling_on_sc=False),
        scratch_types=[
            pltpu.VMEM((VT, BR, VC), jnp.float32),
            pltpu.VMEM((VT, BR, VC), jnp.float32),
            pltpu.VMEM((VT, BR, VC), jnp.float32),
            pltpu.VMEM((V,), jnp.int32),
            pltpu.VMEM((V,), jnp.int32),
            pltpu.VMEM((VT, DR, VC), jnp.float32),
            pltpu.VMEM((VT, DR, VC), jnp.float32),
            pltpu.VMEM((EMB, IN_DIM, L), jnp.float32),
            pltpu.VMEM((EMB, L), jnp.float32),
            pltpu.VMEM((STEPS_PER_DAY * EMB,), jnp.float32),
            pltpu.VMEM((DOW * EMB,), jnp.float32),
            pltpu.SemaphoreType.DMA,
            pltpu.SemaphoreType.DMA,
            pltpu.SemaphoreType.DMA,
            pltpu.SemaphoreType.DMA,
        ],
    )
    out6 = run(x6, wb, bb, todf, dowf)
    return (out6.transpose(0, 1, 3, 5, 2, 4)
                .reshape(B, S, V, IN_DIM * EMB))


# speedup vs baseline: 57.4517x; 2.2808x over previous
import jax
import jax.numpy as jnp
from jax import lax
from jax.experimental import pallas as pl
from jax.experimental.pallas import tpu as pltpu
from jax.experimental.pallas import tpu_sc as plsc

IN_DIM = 3
EMB = 32
STEPS_PER_DAY = 288
DOW = 7
NC = 2
NS = 16
NW = NC * NS
L = 16

B, S, V = 64, 12, 1024
BT, BR = 8, 8
VT, VC = 8, 128
DT, DR = 12, 8
SLABS = S * BT
SLABS_PER_W = SLABS // NW


def _tec_body(x6, wb_hbm, bb_hbm, todf_hbm, dowf_hbm, out6,
              xb0, xb1, xb2, tmap, dmap, blk0, blk1,
              wv, bv, todv, dowv, semx, semt, sem0, sem1):
    wid = lax.axis_index("s") * NC + lax.axis_index("c")

    cp_tab = [
        pltpu.async_copy(wb_hbm, wv, semt),
        pltpu.async_copy(bb_hbm, bv, semt),
        pltpu.async_copy(todf_hbm, todv, semt),
        pltpu.async_copy(dowf_hbm, dowv, semt),
    ]
    for cp in cp_tab:
        cp.wait()

    iota = lax.iota(jnp.int32, L)
    blks = [blk0, blk1]
    sems = [sem0, sem1]

    def slab_body(k, carry):
        slab = wid * SLABS_PER_W + k
        s = slab // BT
        bt = slab % BT

        cpx = [
            pltpu.async_copy(x6.at[s, 0, bt], xb0, semx),
            pltpu.async_copy(x6.at[s, 1, bt], xb1, semx),
            pltpu.async_copy(x6.at[s, 2, bt], xb2, semx),
        ]
        for cp in cpx:
            cp.wait()

        def br_body(br, carry):
            b = bt * BR + br

            @plsc.parallel_loop(0, VT * 8, unroll=2)
            def idx_body(g2):
                vt = g2 // 8
                g = g2 % 8
                x1v = xb1[vt, br, pl.ds(g * L, L)]
                x2v = xb2[vt, br, pl.ds(g * L, L)]
                tmap[pl.ds(g2 * L, L)] = (x1v * float(STEPS_PER_DAY)).astype(jnp.int32)
                dmap[pl.ds(g2 * L, L)] = x2v.astype(jnp.int32)

            for dt in range(DT):
                blk = blks[dt % 2]
                if dt >= 2:
                    pltpu.make_async_copy(blk, out6.at[b, s, dt - 2],
                                          sems[dt % 2]).wait()
                if dt < 4:
                    wrows = [[wv[dt * DR + dr, j] for j in range(IN_DIM)]
                             for dr in range(DR)]
                    brows = [bv[dt * DR + dr] for dr in range(DR)]

                    @plsc.parallel_loop(0, VT * 8, unroll=2)
                    def feat_body(g2):
                        vt = g2 // 8
                        g = g2 % 8
                        x0v = xb0[vt, br, pl.ds(g * L, L)]
                        x1v = xb1[vt, br, pl.ds(g * L, L)]
                        x2v = xb2[vt, br, pl.ds(g * L, L)]
                        fs = [w[0] * x0v + w[1] * x1v + w[2] * x2v + bd
                              for w, bd in zip(wrows, brows)]
                        for dr in range(DR):
                            blk[vt, dr, pl.ds(g * L, L)] = fs[dr]
                else:
                    tab = todv if dt < 8 else dowv
                    imap = tmap if dt < 8 else dmap
                    rows = STEPS_PER_DAY if dt < 8 else DOW
                    dbase = (dt - 4) * DR if dt < 8 else (dt - 8) * DR

                    @plsc.parallel_loop(0, VT * 8, unroll=2)
                    def gat_body(g2):
                        vt = g2 // 8
                        g = g2 % 8
                        iv = imap[pl.ds(g2 * L, L)]
                        rs = [plsc.load_gather(tab, [iv + (dbase + dr) * rows])
                              for dr in range(DR)]
                        for dr in range(DR):
                            blk[vt, dr, pl.ds(g * L, L)] = rs[dr]

                pltpu.async_copy(blk, out6.at[b, s, dt], sems[dt % 2])

            pltpu.make_async_copy(blks[0], out6.at[b, s, DT - 2], sems[0]).wait()
            pltpu.make_async_copy(blks[1], out6.at[b, s, DT - 1], sems[1]).wait()
            return carry
        lax.fori_loop(0, BR, br_body, 0)
        return carry

    lax.fori_loop(0, SLABS_PER_W, slab_body, 0)


def kernel(x, W, b, tod_table, dow_table):
    x6 = (x.transpose(1, 3, 0, 2)
           .reshape(S, IN_DIM, BT, BR, VT, VC)
           .transpose(0, 1, 2, 4, 3, 5))
    wb = jnp.broadcast_to(W[:, :, None], (EMB, IN_DIM, L))
    bb = jnp.broadcast_to(b[:, None], (EMB, L))
    todf = tod_table.T.reshape(STEPS_PER_DAY * EMB)
    dowf = dow_table.T.reshape(DOW * EMB)

    mesh = plsc.VectorSubcoreMesh(core_axis_name="c", subcore_axis_name="s")
    run = pl.kernel(
        _tec_body,
        out_type=jax.ShapeDtypeStruct((B, S, DT, VT, DR, VC), jnp.float32),
        mesh=mesh,
        compiler_params=pltpu.CompilerParams(needs_layout_passes=False,
                                             use_tc_tiling_on_sc=False),
        scratch_types=[
            pltpu.VMEM((VT, BR, VC), jnp.float32),
            pltpu.VMEM((VT, BR, VC), jnp.float32),
            pltpu.VMEM((VT, BR, VC), jnp.float32),
            pltpu.VMEM((V,), jnp.int32),
            pltpu.VMEM((V,), jnp.int32),
            pltpu.VMEM((VT, DR, VC), jnp.float32),
            pltpu.VMEM((VT, DR, VC), jnp.float32),
            pltpu.VMEM((EMB, IN_DIM, L), jnp.float32),
            pltpu.VMEM((EMB, L), jnp.float32),
            pltpu.VMEM((STEPS_PER_DAY * EMB,), jnp.float32),
            pltpu.VMEM((DOW * EMB,), jnp.float32),
            pltpu.SemaphoreType.DMA,
            pltpu.SemaphoreType.DMA,
            pltpu.SemaphoreType.DMA,
            pltpu.SemaphoreType.DMA,
        ],
    )
    out6 = run(x6, wb, bb, todf, dowf)
    return (out6.transpose(0, 1, 3, 5, 2, 4)
                .reshape(B, S, V, IN_DIM * EMB))
